# baseline (speedup 1.0000x reference)
import jax
import jax.numpy as jnp
from jax import lax
from jax.experimental import pallas as pl
from jax.experimental.pallas import tpu as pltpu

B, C, H, W = 32, 96, 28, 28
E, K = 8, 2
HW = H * W
PAD = 32
XPW = HW + 2 * PAD
PR = 9 * C + 8


def _routing_kernel(x_ref, wg_ref, eidx_ref, gp_ref, loss_ref):
    xf = jnp.mean(x_ref[...], axis=2)
    logits = jnp.dot(xf, wg_ref[...],
                     preferred_element_type=jnp.float32)
    eio = lax.broadcasted_iota(jnp.int32, (B, E), 1)
    m1 = jnp.max(logits, axis=1, keepdims=True)
    idx1 = jnp.min(jnp.where(logits == m1, eio, E), axis=1, keepdims=True)
    masked = jnp.where(eio == idx1, -jnp.inf, logits)
    m2 = jnp.max(masked, axis=1, keepdims=True)
    idx2 = jnp.min(jnp.where(masked == m2, eio, E), axis=1, keepdims=True)
    u = jnp.exp(m2 - m1)
    g1 = 1.0 / (1.0 + u)
    g2 = u / (1.0 + u)
    one1 = (eio == idx1).astype(jnp.float32)
    one2 = (eio == idx2).astype(jnp.float32)
    gates = one1 * g1 + one2 * g2
    imp = jnp.sum(gates, axis=0, keepdims=True)
    load = jnp.sum((gates > 0).astype(jnp.float32), axis=0, keepdims=True)

    def cv2(v):
        mv = jnp.mean(v)
        var = jnp.sum((v - mv) ** 2) / (E - 1)
        return var / (mv * mv + 1e-10)

    loss_ref[...] = jnp.reshape((cv2(imp) + cv2(load)) * 0.01, (1, 1))
    eidx_ref[...] = jnp.concatenate([idx1, idx2], axis=1).astype(jnp.int32)
    gp_ref[...] = jnp.concatenate([g1, g2], axis=1)


TPB = 1


def _conv_kernel(eidx_ref, x_ref, w_ref, gp_ref, out_ref, xp, patches):
    g = pl.program_id(0)
    wcol = lax.broadcasted_iota(jnp.int32, (1, HW), 1) % W
    rio = lax.broadcasted_iota(jnp.int32, (8, HW), 0)
    for t in range(TPB):
        xv = x_ref[t].astype(jnp.bfloat16)
        xp[:, :PAD] = jnp.zeros((C, PAD), jnp.bfloat16)
        xp[:, PAD + HW:] = jnp.zeros((C, PAD), jnp.bfloat16)
        xp[:, PAD:PAD + HW] = xv
        for j in range(9):
            dh, dw = j // 3 - 1, j % 3 - 1
            s = dh * W + dw
            xs = xp[:, PAD + s:PAD + s + HW]
            if dw == 1:
                xs = jnp.where(wcol == W - 1, jnp.bfloat16(0), xs)
            elif dw == -1:
                xs = jnp.where(wcol == 0, jnp.bfloat16(0), xs)
            patches[j * C:(j + 1) * C, t * HW:(t + 1) * HW] = xs
        patches[9 * C:, t * HW:(t + 1) * HW] = (
            jnp.where(rio == 0, 1.0, 0.0).astype(jnp.bfloat16))
    for t in range(TPB):
        b = g * TPB + t
        pm = patches[:, t * HW:(t + 1) * HW]
        gp = gp_ref[pl.ds(b, 1), :]
        y = jnp.zeros((C, HW), jnp.float32)
        for k in range(K):
            e = eidx_ref[b, k]
            acc = jnp.dot(w_ref[e], pm,
                          preferred_element_type=jnp.float32)
            y = y + jnp.maximum(acc, 0.0) * gp[:, k:k + 1]
        out_ref[t] = y.reshape(C, H, W)


def kernel(x, w_gate, conv_w, conv_b, bn_gamma, bn_beta, bn_mean, bn_var):
    x3 = x.reshape(B, C, HW)
    scale = bn_gamma / jnp.sqrt(bn_var + 1e-5)
    shift = (conv_b - bn_mean) * scale + bn_beta
    wt = conv_w.reshape(E, C, C, 9).transpose(0, 1, 3, 2).reshape(E, C, 9 * C)
    wt = jnp.concatenate(
        [wt * scale[:, :, None], shift[:, :, None],
         jnp.zeros((E, C, PR - 9 * C - 1), jnp.float32)],
        axis=2).astype(jnp.bfloat16)

    eidx, gp, loss = pl.pallas_call(
        _routing_kernel,
        grid=(1,),
        in_specs=[
            pl.BlockSpec((B, C, HW), lambda i: (0, 0, 0)),
            pl.BlockSpec((C, E), lambda i: (0, 0)),
        ],
        out_specs=[
            pl.BlockSpec((B, K), lambda i: (0, 0)),
            pl.BlockSpec((B, K), lambda i: (0, 0)),
            pl.BlockSpec((1, 1), lambda i: (0, 0)),
        ],
        out_shape=[
            jax.ShapeDtypeStruct((B, K), jnp.int32),
            jax.ShapeDtypeStruct((B, K), jnp.float32),
            jax.ShapeDtypeStruct((1, 1), jnp.float32),
        ],
    )(x3, w_gate)

    combined = pl.pallas_call(
        _conv_kernel,
        grid_spec=pltpu.PrefetchScalarGridSpec(
            num_scalar_prefetch=1,
            grid=(B // TPB,),
            in_specs=[
                pl.BlockSpec((TPB, C, HW), lambda b, eref: (b, 0, 0)),
                pl.BlockSpec((E, C, PR), lambda b, eref: (0, 0, 0)),
                pl.BlockSpec((B, K), lambda b, eref: (0, 0)),
            ],
            out_specs=pl.BlockSpec((TPB, C, H, W), lambda b, eref: (b, 0, 0, 0)),
            scratch_shapes=[
                pltpu.VMEM((C, XPW), jnp.bfloat16),
                pltpu.VMEM((PR, TPB * HW), jnp.bfloat16),
            ],
        ),
        out_shape=jax.ShapeDtypeStruct((B, C, H, W), jnp.float32),
        compiler_params=pltpu.CompilerParams(
            dimension_semantics=("parallel",),
        ),
    )(eidx, x3, wt, gp)

    return combined, loss[0, 0]


# speedup vs baseline: 1.4446x; 1.4446x over previous
import jax
import jax.numpy as jnp
from jax import lax
from jax.experimental import pallas as pl
from jax.experimental.pallas import tpu as pltpu

B, C, H, W = 32, 96, 28, 28
E, K = 8, 2
HW = H * W
PAD = 32
XPW = HW + 2 * PAD
PR = 9 * C + 8


def _routing_kernel(x_ref, wg_ref, eidx_ref, gp_ref, loss_ref):
    xf = jnp.mean(x_ref[...], axis=2)
    logits = jnp.dot(xf, wg_ref[...],
                     preferred_element_type=jnp.float32)
    eio = lax.broadcasted_iota(jnp.int32, (B, E), 1)
    m1 = jnp.max(logits, axis=1, keepdims=True)
    idx1 = jnp.min(jnp.where(logits == m1, eio, E), axis=1, keepdims=True)
    masked = jnp.where(eio == idx1, -jnp.inf, logits)
    m2 = jnp.max(masked, axis=1, keepdims=True)
    idx2 = jnp.min(jnp.where(masked == m2, eio, E), axis=1, keepdims=True)
    u = jnp.exp(m2 - m1)
    g1 = 1.0 / (1.0 + u)
    g2 = u / (1.0 + u)
    one1 = (eio == idx1).astype(jnp.float32)
    one2 = (eio == idx2).astype(jnp.float32)
    gates = one1 * g1 + one2 * g2
    imp = jnp.sum(gates, axis=0, keepdims=True)
    load = jnp.sum((gates > 0).astype(jnp.float32), axis=0, keepdims=True)

    def cv2(v):
        mv = jnp.mean(v)
        var = jnp.sum((v - mv) ** 2) / (E - 1)
        return var / (mv * mv + 1e-10)

    loss_ref[...] = jnp.reshape((cv2(imp) + cv2(load)) * 0.01, (1, 1))
    eidx_ref[...] = jnp.concatenate([idx1, idx2], axis=1).astype(jnp.int32)
    gp_ref[...] = jnp.concatenate([g1, g2], axis=1)


TPB = 4


def _conv_kernel(eidx_ref, x_ref, w_ref, gp_ref, out_ref, xp, patches):
    g = pl.program_id(0)
    wcol = lax.broadcasted_iota(jnp.int32, (1, HW), 1) % W
    rio = lax.broadcasted_iota(jnp.int32, (8, HW), 0)
    ones_row = jnp.where(rio == 0, 1.0, 0.0).astype(jnp.bfloat16)
    for t in range(TPB):
        xv = x_ref[t]
        xp[:, :PAD] = jnp.zeros((C, PAD), jnp.bfloat16)
        xp[:, PAD + HW:] = jnp.zeros((C, PAD), jnp.bfloat16)
        xp[:, PAD:PAD + HW] = xv
        for j in range(9):
            dh, dw = j // 3 - 1, j % 3 - 1
            s = dh * W + dw
            xs = xp[:, PAD + s:PAD + s + HW]
            if dw == 1:
                xs = jnp.where(wcol == W - 1, jnp.bfloat16(0), xs)
            elif dw == -1:
                xs = jnp.where(wcol == 0, jnp.bfloat16(0), xs)
            patches[t, j * C:(j + 1) * C, :] = xs
        patches[t, 9 * C:, :] = ones_row
    for t in range(TPB):
        b = g * TPB + t
        pm = patches[t]
        gp = gp_ref[pl.ds(b, 1), :]
        y = jnp.zeros((C, HW), jnp.float32)
        for k in range(K):
            e = eidx_ref[b, k]
            acc = jnp.dot(w_ref[e], pm,
                          preferred_element_type=jnp.float32)
            y = y + jnp.maximum(acc, 0.0) * gp[:, k:k + 1]
        out_ref[t] = y


def kernel(x, w_gate, conv_w, conv_b, bn_gamma, bn_beta, bn_mean, bn_var):
    x3 = x.reshape(B, C, HW)
    scale = bn_gamma / jnp.sqrt(bn_var + 1e-5)
    shift = (conv_b - bn_mean) * scale + bn_beta
    wt = conv_w.reshape(E, C, C, 9).transpose(0, 1, 3, 2).reshape(E, C, 9 * C)
    wt = jnp.concatenate(
        [wt * scale[:, :, None], shift[:, :, None],
         jnp.zeros((E, C, PR - 9 * C - 1), jnp.float32)],
        axis=2).astype(jnp.bfloat16)

    eidx, gp, loss = pl.pallas_call(
        _routing_kernel,
        grid=(1,),
        in_specs=[
            pl.BlockSpec((B, C, HW), lambda i: (0, 0, 0)),
            pl.BlockSpec((C, E), lambda i: (0, 0)),
        ],
        out_specs=[
            pl.BlockSpec((B, K), lambda i: (0, 0)),
            pl.BlockSpec((B, K), lambda i: (0, 0)),
            pl.BlockSpec((1, 1), lambda i: (0, 0)),
        ],
        out_shape=[
            jax.ShapeDtypeStruct((B, K), jnp.int32),
            jax.ShapeDtypeStruct((B, K), jnp.float32),
            jax.ShapeDtypeStruct((1, 1), jnp.float32),
        ],
    )(x3, w_gate)

    combined = pl.pallas_call(
        _conv_kernel,
        grid_spec=pltpu.PrefetchScalarGridSpec(
            num_scalar_prefetch=1,
            grid=(B // TPB,),
            in_specs=[
                pl.BlockSpec((TPB, C, HW), lambda b, eref: (b, 0, 0)),
                pl.BlockSpec((E, C, PR), lambda b, eref: (0, 0, 0)),
                pl.BlockSpec((B, K), lambda b, eref: (0, 0)),
            ],
            out_specs=pl.BlockSpec((TPB, C, HW), lambda b, eref: (b, 0, 0)),
            scratch_shapes=[
                pltpu.VMEM((C, XPW), jnp.bfloat16),
                pltpu.VMEM((TPB, PR, HW), jnp.bfloat16),
            ],
        ),
        out_shape=jax.ShapeDtypeStruct((B, C, HW), jnp.float32),
        compiler_params=pltpu.CompilerParams(
            dimension_semantics=("parallel",),
        ),
    )(eidx, x3.astype(jnp.bfloat16), wt, gp)

    return combined.reshape(B, C, H, W), loss[0, 0]


# speedup vs baseline: 1.5700x; 1.0868x over previous
import jax
import jax.numpy as jnp
from jax import lax
from jax.experimental import pallas as pl
from jax.experimental.pallas import tpu as pltpu

B, C, H, W = 32, 96, 28, 28
E, K = 8, 2
HW = H * W
PAD = 32
XPW = HW + 2 * PAD
PR = 9 * C + 8


def _routing_kernel(x_ref, wg_ref, eidx_ref, gp_ref, loss_ref):
    xf = jnp.mean(x_ref[...], axis=2)
    logits = jnp.dot(xf, wg_ref[...],
                     preferred_element_type=jnp.float32)
    eio = lax.broadcasted_iota(jnp.int32, (B, E), 1)
    m1 = jnp.max(logits, axis=1, keepdims=True)
    idx1 = jnp.min(jnp.where(logits == m1, eio, E), axis=1, keepdims=True)
    masked = jnp.where(eio == idx1, -jnp.inf, logits)
    m2 = jnp.max(masked, axis=1, keepdims=True)
    idx2 = jnp.min(jnp.where(masked == m2, eio, E), axis=1, keepdims=True)
    u = jnp.exp(m2 - m1)
    g1 = 1.0 / (1.0 + u)
    g2 = u / (1.0 + u)
    one1 = (eio == idx1).astype(jnp.float32)
    one2 = (eio == idx2).astype(jnp.float32)
    gates = one1 * g1 + one2 * g2
    imp = jnp.sum(gates, axis=0, keepdims=True)
    load = jnp.sum((gates > 0).astype(jnp.float32), axis=0, keepdims=True)

    def cv2(v):
        mv = jnp.mean(v)
        var = jnp.sum((v - mv) ** 2) / (E - 1)
        return var / (mv * mv + 1e-10)

    loss_ref[...] = jnp.reshape((cv2(imp) + cv2(load)) * 0.01, (1, 1))
    eidx_ref[...] = jnp.concatenate([idx1, idx2], axis=1).astype(jnp.int32)
    gp_ref[...] = jnp.concatenate([g1, g2], axis=1)


TPB = 4


def _conv_kernel(eidx_ref, x_ref, w_ref, gp_ref, out_ref, xp, patches):
    g = pl.program_id(0)
    wcol = lax.broadcasted_iota(jnp.int32, (1, HW), 1) % W
    rio = lax.broadcasted_iota(jnp.int32, (8, HW), 0)
    ones_row = jnp.where(rio == 0, 1.0, 0.0).astype(jnp.bfloat16)
    for t in range(TPB):
        xv = x_ref[t].astype(jnp.bfloat16)
        xp[:, :PAD] = jnp.zeros((C, PAD), jnp.bfloat16)
        xp[:, PAD + HW:] = jnp.zeros((C, PAD), jnp.bfloat16)
        xp[:, PAD:PAD + HW] = xv
        for j in range(9):
            dh, dw = j // 3 - 1, j % 3 - 1
            s = dh * W + dw
            xs = xp[:, PAD + s:PAD + s + HW]
            if dw == 1:
                xs = jnp.where(wcol == W - 1, jnp.bfloat16(0), xs)
            elif dw == -1:
                xs = jnp.where(wcol == 0, jnp.bfloat16(0), xs)
            patches[t, j * C:(j + 1) * C, :] = xs
        patches[t, 9 * C:, :] = ones_row
    for t in range(TPB):
        b = g * TPB + t
        pm = patches[t]
        gp = gp_ref[pl.ds(b, 1), :]
        y = jnp.zeros((C, HW), jnp.float32)
        for k in range(K):
            e = eidx_ref[b, k]
            acc = jnp.dot(w_ref[e], pm,
                          preferred_element_type=jnp.float32)
            y = y + jnp.maximum(acc, 0.0) * gp[:, k:k + 1]
        out_ref[t] = y


def kernel(x, w_gate, conv_w, conv_b, bn_gamma, bn_beta, bn_mean, bn_var):
    x3 = x.reshape(B, C, HW)
    scale = bn_gamma / jnp.sqrt(bn_var + 1e-5)
    shift = (conv_b - bn_mean) * scale + bn_beta
    wt = conv_w.reshape(E, C, C, 9).transpose(0, 1, 3, 2).reshape(E, C, 9 * C)
    wt = jnp.concatenate(
        [wt * scale[:, :, None], shift[:, :, None],
         jnp.zeros((E, C, PR - 9 * C - 1), jnp.float32)],
        axis=2).astype(jnp.bfloat16)

    eidx, gp, loss = pl.pallas_call(
        _routing_kernel,
        grid=(1,),
        in_specs=[
            pl.BlockSpec((B, C, HW), lambda i: (0, 0, 0)),
            pl.BlockSpec((C, E), lambda i: (0, 0)),
        ],
        out_specs=[
            pl.BlockSpec((B, K), lambda i: (0, 0)),
            pl.BlockSpec((B, K), lambda i: (0, 0)),
            pl.BlockSpec((1, 1), lambda i: (0, 0)),
        ],
        out_shape=[
            jax.ShapeDtypeStruct((B, K), jnp.int32),
            jax.ShapeDtypeStruct((B, K), jnp.float32),
            jax.ShapeDtypeStruct((1, 1), jnp.float32),
        ],
    )(x3, w_gate)

    combined = pl.pallas_call(
        _conv_kernel,
        grid_spec=pltpu.PrefetchScalarGridSpec(
            num_scalar_prefetch=1,
            grid=(B // TPB,),
            in_specs=[
                pl.BlockSpec((TPB, C, HW), lambda b, eref: (b, 0, 0)),
                pl.BlockSpec((E, C, PR), lambda b, eref: (0, 0, 0)),
                pl.BlockSpec((B, K), lambda b, eref: (0, 0)),
            ],
            out_specs=pl.BlockSpec((TPB, C, HW), lambda b, eref: (b, 0, 0)),
            scratch_shapes=[
                pltpu.VMEM((C, XPW), jnp.bfloat16),
                pltpu.VMEM((TPB, PR, HW), jnp.bfloat16),
            ],
        ),
        out_shape=jax.ShapeDtypeStruct((B, C, HW), jnp.float32),
        compiler_params=pltpu.CompilerParams(
            dimension_semantics=("parallel",),
        ),
    )(eidx, x3, wt, gp)

    return combined.reshape(B, C, H, W), loss[0, 0]


# speedup vs baseline: 1.5799x; 1.0063x over previous
import jax
import jax.numpy as jnp
from jax import lax
from jax.experimental import pallas as pl
from jax.experimental.pallas import tpu as pltpu

B, C, H, W = 32, 96, 28, 28
E, K = 8, 2
HW = H * W
PAD = 32
XPW = HW + 2 * PAD
PR = 9 * C + 8


def _routing_kernel(x_ref, wg_ref, eidx_ref, gp_ref, loss_ref):
    xf = jnp.mean(x_ref[...], axis=2)
    logits = jnp.dot(xf, wg_ref[...],
                     preferred_element_type=jnp.float32)
    eio = lax.broadcasted_iota(jnp.int32, (B, E), 1)
    m1 = jnp.max(logits, axis=1, keepdims=True)
    idx1 = jnp.min(jnp.where(logits == m1, eio, E), axis=1, keepdims=True)
    masked = jnp.where(eio == idx1, -jnp.inf, logits)
    m2 = jnp.max(masked, axis=1, keepdims=True)
    idx2 = jnp.min(jnp.where(masked == m2, eio, E), axis=1, keepdims=True)
    u = jnp.exp(m2 - m1)
    g1 = 1.0 / (1.0 + u)
    g2 = u / (1.0 + u)
    one1 = (eio == idx1).astype(jnp.float32)
    one2 = (eio == idx2).astype(jnp.float32)
    gates = one1 * g1 + one2 * g2
    imp = jnp.sum(gates, axis=0, keepdims=True)
    load = jnp.sum((gates > 0).astype(jnp.float32), axis=0, keepdims=True)

    def cv2(v):
        mv = jnp.mean(v)
        var = jnp.sum((v - mv) ** 2) / (E - 1)
        return var / (mv * mv + 1e-10)

    loss_ref[...] = jnp.reshape((cv2(imp) + cv2(load)) * 0.01, (1, 1))
    eidx_ref[...] = jnp.concatenate([idx1, idx2], axis=1).astype(jnp.int32)
    gp_ref[...] = jnp.concatenate([g1, g2], axis=1)


TPB = 8


def _conv_kernel(eidx_ref, x_ref, w_ref, gp_ref, out_ref, xp, patches):
    g = pl.program_id(0)
    wcol = lax.broadcasted_iota(jnp.int32, (1, HW), 1) % W
    rio = lax.broadcasted_iota(jnp.int32, (8, HW), 0)
    ones_row = jnp.where(rio == 0, 1.0, 0.0).astype(jnp.bfloat16)
    for t in range(TPB):
        xv = x_ref[t].astype(jnp.bfloat16)
        xp[:, :PAD] = jnp.zeros((C, PAD), jnp.bfloat16)
        xp[:, PAD + HW:] = jnp.zeros((C, PAD), jnp.bfloat16)
        xp[:, PAD:PAD + HW] = xv
        for j in range(9):
            dh, dw = j // 3 - 1, j % 3 - 1
            s = dh * W + dw
            xs = xp[:, PAD + s:PAD + s + HW]
            if dw == 1:
                xs = jnp.where(wcol == W - 1, jnp.bfloat16(0), xs)
            elif dw == -1:
                xs = jnp.where(wcol == 0, jnp.bfloat16(0), xs)
            patches[t, j * C:(j + 1) * C, :] = xs
        patches[t, 9 * C:, :] = ones_row
    for t in range(TPB):
        b = g * TPB + t
        pm = patches[t]
        gp = gp_ref[pl.ds(b, 1), :]
        y = jnp.zeros((C, HW), jnp.float32)
        for k in range(K):
            e = eidx_ref[b, k]
            acc = jnp.dot(w_ref[e], pm,
                          preferred_element_type=jnp.float32)
            y = y + jnp.maximum(acc, 0.0) * gp[:, k:k + 1]
        out_ref[t] = y


def kernel(x, w_gate, conv_w, conv_b, bn_gamma, bn_beta, bn_mean, bn_var):
    x3 = x.reshape(B, C, HW)
    scale = bn_gamma / jnp.sqrt(bn_var + 1e-5)
    shift = (conv_b - bn_mean) * scale + bn_beta
    wt = conv_w.reshape(E, C, C, 9).transpose(0, 1, 3, 2).reshape(E, C, 9 * C)
    wt = jnp.concatenate(
        [wt * scale[:, :, None], shift[:, :, None],
         jnp.zeros((E, C, PR - 9 * C - 1), jnp.float32)],
        axis=2).astype(jnp.bfloat16)

    eidx, gp, loss = pl.pallas_call(
        _routing_kernel,
        grid=(1,),
        in_specs=[
            pl.BlockSpec((B, C, HW), lambda i: (0, 0, 0)),
            pl.BlockSpec((C, E), lambda i: (0, 0)),
        ],
        out_specs=[
            pl.BlockSpec((B, K), lambda i: (0, 0)),
            pl.BlockSpec((B, K), lambda i: (0, 0)),
            pl.BlockSpec((1, 1), lambda i: (0, 0)),
        ],
        out_shape=[
            jax.ShapeDtypeStruct((B, K), jnp.int32),
            jax.ShapeDtypeStruct((B, K), jnp.float32),
            jax.ShapeDtypeStruct((1, 1), jnp.float32),
        ],
    )(x3, w_gate)

    combined = pl.pallas_call(
        _conv_kernel,
        grid_spec=pltpu.PrefetchScalarGridSpec(
            num_scalar_prefetch=1,
            grid=(B // TPB,),
            in_specs=[
                pl.BlockSpec((TPB, C, HW), lambda b, eref: (b, 0, 0)),
                pl.BlockSpec((E, C, PR), lambda b, eref: (0, 0, 0)),
                pl.BlockSpec((B, K), lambda b, eref: (0, 0)),
            ],
            out_specs=pl.BlockSpec((TPB, C, HW), lambda b, eref: (b, 0, 0)),
            scratch_shapes=[
                pltpu.VMEM((C, XPW), jnp.bfloat16),
                pltpu.VMEM((TPB, PR, HW), jnp.bfloat16),
            ],
        ),
        out_shape=jax.ShapeDtypeStruct((B, C, HW), jnp.float32),
        compiler_params=pltpu.CompilerParams(
            dimension_semantics=("parallel",),
        ),
    )(eidx, x3, wt, gp)

    return combined.reshape(B, C, H, W), loss[0, 0]
